# speedup vs baseline: 2.0827x; 2.0827x over previous
import functools

import jax
import jax.numpy as jnp
from jax import lax
from jax.experimental import pallas as pl
from jax.experimental.pallas import tpu as pltpu
from jax.experimental.pallas import tpu_sc as plsc

N = 10000
L = 16
E = 320000
V = 100000
D = 128
B = 64

NC = 2
NS = 16
NW = NC * NS
CHUNK = 128
ACC_ROWS = 10240
ROWS_PER_TILE = ACC_ROWS // NS

EP0 = 163840
EP1 = 327680

_mesh = plsc.VectorSubcoreMesh(core_axis_name="c", subcore_axis_name="s")


def _make_scatter(ep):
    per_tile = ep // NW
    n_chunks = per_tile // CHUNK

    def body(table_hbm, src_hbm, dst_hbm, zeros_hbm, out_hbm,
             src_v, dst_v, rows_v, acc_sh, sem):
        c = lax.axis_index("c")
        s = lax.axis_index("s")
        wid = c * NS + s

        pltpu.sync_copy(zeros_hbm, acc_sh.at[pl.ds(s * ROWS_PER_TILE, ROWS_PER_TILE)])
        plsc.subcore_barrier()

        base = wid * per_tile

        @pl.loop(0, n_chunks)
        def _(ci):
            off = base + ci * CHUNK
            pltpu.sync_copy(src_hbm.at[pl.ds(off, CHUNK)], src_v)
            pltpu.sync_copy(dst_hbm.at[pl.ds(off, CHUNK)], dst_v)
            pltpu.async_copy(table_hbm.at[src_v], rows_v, sem).wait()
            pltpu.sync_copy(rows_v, acc_sh.at[dst_v], add=True)

        plsc.subcore_barrier()
        pltpu.sync_copy(
            acc_sh.at[pl.ds(s * ROWS_PER_TILE, ROWS_PER_TILE)],
            out_hbm.at[pl.ds(c * ACC_ROWS + s * ROWS_PER_TILE, ROWS_PER_TILE)])

    return pl.kernel(
        body,
        out_type=jax.ShapeDtypeStruct((NC * ACC_ROWS, D), jnp.float32),
        mesh=_mesh,
        scratch_types=[
            pltpu.VMEM((CHUNK,), jnp.int32),
            pltpu.VMEM((CHUNK,), jnp.int32),
            pltpu.VMEM((CHUNK, D), jnp.float32),
            pltpu.VMEM_SHARED((ACC_ROWS, D), jnp.float32),
            pltpu.SemaphoreType.DMA,
        ],
    )


_scatter_emb = _make_scatter(EP0)
_scatter_edge = _make_scatter(EP1)

_ROW_BLK = 1000
_GRID = N // _ROW_BLK


def _combine_body(p0_ref, p1_ref, o_ref):
    o_ref[...] = p0_ref[0] + p1_ref[0]


def _tc_combine(p):
    return pl.pallas_call(
        _combine_body,
        grid=(_GRID,),
        in_specs=[
            pl.BlockSpec((1, _ROW_BLK, D), lambda i: (0, i, 0)),
            pl.BlockSpec((1, _ROW_BLK, D), lambda i: (1, i, 0)),
        ],
        out_specs=pl.BlockSpec((_ROW_BLK, D), lambda i: (i, 0)),
        out_shape=jax.ShapeDtypeStruct((N, D), jnp.float32),
    )(p, p)


def _layer_body(h_ref, q0_ref, q1_ref, w_ref, b_ref, o_ref):
    z = h_ref[...] + q0_ref[0] + q1_ref[0]
    y = jnp.dot(z, w_ref[...], preferred_element_type=jnp.float32) + b_ref[...]
    o_ref[...] = jnp.maximum(y, 0.0)


def _tc_layer(h, q, w, b):
    return pl.pallas_call(
        _layer_body,
        grid=(_GRID,),
        in_specs=[
            pl.BlockSpec((_ROW_BLK, D), lambda i: (i, 0)),
            pl.BlockSpec((1, _ROW_BLK, D), lambda i: (0, i, 0)),
            pl.BlockSpec((1, _ROW_BLK, D), lambda i: (1, i, 0)),
            pl.BlockSpec((D, D), lambda i: (0, 0)),
            pl.BlockSpec((1, D), lambda i: (0, 0)),
        ],
        out_specs=pl.BlockSpec((_ROW_BLK, D), lambda i: (i, 0)),
        out_shape=jax.ShapeDtypeStruct((N, D), jnp.float32),
    )(h, q, q, w, b.reshape(1, D))


def _pool_body(h_ref, r0_ref, r1_ref, w_ref, b_ref, batch_ref, o_ref):
    z = h_ref[...] + r0_ref[0] + r1_ref[0]
    h2 = jnp.maximum(
        jnp.dot(z, w_ref[...], preferred_element_type=jnp.float32) + b_ref[...], 0.0)
    bvec = batch_ref[0, 0, :]
    onehot = (bvec[:, None] == lax.broadcasted_iota(jnp.int32, (_ROW_BLK, B), 1)
              ).astype(jnp.float32)
    contrib = lax.dot_general(onehot, h2, (((0,), (0,)), ((), ())),
                              preferred_element_type=jnp.float32)

    @pl.when(pl.program_id(0) == 0)
    def _():
        o_ref[...] = jnp.zeros_like(o_ref)

    o_ref[...] += contrib


def _tc_pool(h, r, w, b, batch3):
    return pl.pallas_call(
        _pool_body,
        grid=(_GRID,),
        in_specs=[
            pl.BlockSpec((_ROW_BLK, D), lambda i: (i, 0)),
            pl.BlockSpec((1, _ROW_BLK, D), lambda i: (0, i, 0)),
            pl.BlockSpec((1, _ROW_BLK, D), lambda i: (1, i, 0)),
            pl.BlockSpec((D, D), lambda i: (0, 0)),
            pl.BlockSpec((1, D), lambda i: (0, 0)),
            pl.BlockSpec((1, 1, _ROW_BLK), lambda i: (i, 0, 0)),
        ],
        out_specs=pl.BlockSpec((B, D), lambda i: (0, 0)),
        out_shape=jax.ShapeDtypeStruct((B, D), jnp.float32),
    )(h, r, r, w, b.reshape(1, D), batch3)


def kernel(x, edge_index, batch, emb_table, W0, b0, W1, b1):
    x = x.astype(jnp.int32)
    src0 = jnp.concatenate(
        [x.reshape(-1), jnp.zeros((EP0 - N * L,), jnp.int32)])
    dst0 = jnp.concatenate(
        [jnp.repeat(jnp.arange(N, dtype=jnp.int32), L),
         jnp.full((EP0 - N * L,), N, jnp.int32)])
    src1 = jnp.concatenate(
        [edge_index[0].astype(jnp.int32), jnp.zeros((EP1 - E,), jnp.int32)])
    dst1 = jnp.concatenate(
        [edge_index[1].astype(jnp.int32), jnp.full((EP1 - E,), N, jnp.int32)])
    zeros_blk = jnp.zeros((ROWS_PER_TILE, D), jnp.float32)
    batch3 = batch.astype(jnp.int32).reshape(_GRID, 1, _ROW_BLK)

    p = _scatter_emb(emb_table, src0, dst0, zeros_blk).reshape(NC, ACC_ROWS, D)
    h0 = _tc_combine(p)
    q = _scatter_edge(h0, src1, dst1, zeros_blk).reshape(NC, ACC_ROWS, D)
    h1 = _tc_layer(h0, q, W0, b0)
    r = _scatter_edge(h1, src1, dst1, zeros_blk).reshape(NC, ACC_ROWS, D)
    return _tc_pool(h1, r, W1, b1, batch3)


# speedup vs baseline: 2.6355x; 1.2654x over previous
import functools

import jax
import jax.numpy as jnp
from jax import lax
from jax.experimental import pallas as pl
from jax.experimental.pallas import tpu as pltpu
from jax.experimental.pallas import tpu_sc as plsc

N = 10000
L = 16
E = 320000
V = 100000
D = 128
B = 64

NC = 2
NS = 16
NW = NC * NS
CHUNK = 128

NHALF = N // NC
EMB_ACC_ROWS = 5120
EMB_PER_TILE = EMB_ACC_ROWS
EMB_PER_SC = NS * EMB_PER_TILE
EMB_ROWS_PER_TILE = EMB_ACC_ROWS // NS
EMB_NCHUNKS = EMB_PER_TILE // CHUNK

ACC_ROWS = 10240
ROWS_PER_TILE = ACC_ROWS // NS
EP1 = 327680
EDGE_PER_TILE = EP1 // NW
EDGE_NCHUNKS = EDGE_PER_TILE // CHUNK

_mesh = plsc.VectorSubcoreMesh(core_axis_name="c", subcore_axis_name="s")


def _gather_scatter_loop(table_hbm, src_hbm, dst_hbm, acc_sh,
                         src_v, dst_v, rows_v, sems, idx_base, n_chunks):
    def _load_idx(b, ci):
        off = idx_base + ci * CHUNK
        pltpu.sync_copy(src_hbm.at[pl.ds(off, CHUNK)], src_v.at[b])
        pltpu.sync_copy(dst_hbm.at[pl.ds(off, CHUNK)], dst_v.at[b])

    def _gather(b):
        return pltpu.make_async_copy(table_hbm.at[src_v.at[b]],
                                     rows_v.at[b], sems[b])

    for b in range(2):
        _load_idx(b, b)
        _gather(b).start()

    @pl.loop(0, n_chunks // 2)
    def _(t):
        for b in range(2):
            ci = 2 * t + b
            _gather(b).wait()
            pltpu.sync_copy(rows_v.at[b], acc_sh.at[dst_v.at[b]], add=True)

            @pl.when(t < n_chunks // 2 - 1)
            def _():
                _load_idx(b, ci + 2)
                _gather(b).start()


def _emb_body(table_hbm, src_hbm, dst_hbm, zeros_hbm, out_hbm,
              src_v, dst_v, rows_v, acc_sh, sem0, sem1):
    c = lax.axis_index("c")
    s = lax.axis_index("s")

    pltpu.sync_copy(zeros_hbm.at[pl.ds(0, EMB_ROWS_PER_TILE)],
                    acc_sh.at[pl.ds(s * EMB_ROWS_PER_TILE, EMB_ROWS_PER_TILE)])
    plsc.subcore_barrier()

    idx_base = c * EMB_PER_SC + s * EMB_PER_TILE
    _gather_scatter_loop(table_hbm, src_hbm, dst_hbm, acc_sh,
                         src_v, dst_v, rows_v, (sem0, sem1),
                         idx_base, EMB_NCHUNKS)

    plsc.subcore_barrier()
    local = s * EMB_ROWS_PER_TILE
    last_rows = NHALF - (NS - 1) * EMB_ROWS_PER_TILE

    @pl.when(s < NS - 1)
    def _():
        pltpu.sync_copy(
            acc_sh.at[pl.ds(local, EMB_ROWS_PER_TILE)],
            out_hbm.at[pl.ds(c * NHALF + local, EMB_ROWS_PER_TILE)])

    @pl.when(s == NS - 1)
    def _():
        pltpu.sync_copy(
            acc_sh.at[pl.ds(local, last_rows)],
            out_hbm.at[pl.ds(c * NHALF + local, last_rows)])


_emb_kernel = pl.kernel(
    _emb_body,
    out_type=jax.ShapeDtypeStruct((N, D), jnp.float32),
    mesh=_mesh,
    scratch_types=[
        pltpu.VMEM((2, CHUNK), jnp.int32),
        pltpu.VMEM((2, CHUNK), jnp.int32),
        pltpu.VMEM((2, CHUNK, D), jnp.float32),
        pltpu.VMEM_SHARED((EMB_ACC_ROWS, D), jnp.float32),
        pltpu.SemaphoreType.DMA,
        pltpu.SemaphoreType.DMA,
    ],
)


def _edge_body(table_hbm, src_hbm, dst_hbm, zeros_hbm, out_hbm,
               src_v, dst_v, rows_v, acc_sh, sem0, sem1):
    c = lax.axis_index("c")
    s = lax.axis_index("s")
    wid = c * NS + s

    pltpu.sync_copy(zeros_hbm,
                    acc_sh.at[pl.ds(s * ROWS_PER_TILE, ROWS_PER_TILE)])
    plsc.subcore_barrier()

    _gather_scatter_loop(table_hbm, src_hbm, dst_hbm, acc_sh,
                         src_v, dst_v, rows_v, (sem0, sem1),
                         wid * EDGE_PER_TILE, EDGE_NCHUNKS)

    plsc.subcore_barrier()
    pltpu.sync_copy(
        acc_sh.at[pl.ds(s * ROWS_PER_TILE, ROWS_PER_TILE)],
        out_hbm.at[pl.ds(c * ACC_ROWS + s * ROWS_PER_TILE, ROWS_PER_TILE)])


_edge_kernel = pl.kernel(
    _edge_body,
    out_type=jax.ShapeDtypeStruct((NC * ACC_ROWS, D), jnp.float32),
    mesh=_mesh,
    scratch_types=[
        pltpu.VMEM((2, CHUNK), jnp.int32),
        pltpu.VMEM((2, CHUNK), jnp.int32),
        pltpu.VMEM((2, CHUNK, D), jnp.float32),
        pltpu.VMEM_SHARED((ACC_ROWS, D), jnp.float32),
        pltpu.SemaphoreType.DMA,
        pltpu.SemaphoreType.DMA,
    ],
)

_ROW_BLK = 1000
_GRID = N // _ROW_BLK


def _layer_body(h_ref, q0_ref, q1_ref, w_ref, b_ref, o_ref):
    z = h_ref[...] + q0_ref[0] + q1_ref[0]
    y = jnp.dot(z, w_ref[...], preferred_element_type=jnp.float32) + b_ref[...]
    o_ref[...] = jnp.maximum(y, 0.0)


def _tc_layer(h, q, w, b):
    return pl.pallas_call(
        _layer_body,
        grid=(_GRID,),
        in_specs=[
            pl.BlockSpec((_ROW_BLK, D), lambda i: (i, 0)),
            pl.BlockSpec((1, _ROW_BLK, D), lambda i: (0, i, 0)),
            pl.BlockSpec((1, _ROW_BLK, D), lambda i: (1, i, 0)),
            pl.BlockSpec((D, D), lambda i: (0, 0)),
            pl.BlockSpec((1, D), lambda i: (0, 0)),
        ],
        out_specs=pl.BlockSpec((_ROW_BLK, D), lambda i: (i, 0)),
        out_shape=jax.ShapeDtypeStruct((N, D), jnp.float32),
    )(h, q, q, w, b.reshape(1, D))


def _pool_body(h_ref, r0_ref, r1_ref, w_ref, b_ref, batch_ref, o_ref):
    z = h_ref[...] + r0_ref[0] + r1_ref[0]
    h2 = jnp.maximum(
        jnp.dot(z, w_ref[...], preferred_element_type=jnp.float32) + b_ref[...], 0.0)
    bvec = batch_ref[0, 0, :]
    onehot = (bvec[:, None] == lax.broadcasted_iota(jnp.int32, (_ROW_BLK, B), 1)
              ).astype(jnp.float32)
    contrib = lax.dot_general(onehot, h2, (((0,), (0,)), ((), ())),
                              preferred_element_type=jnp.float32)

    @pl.when(pl.program_id(0) == 0)
    def _():
        o_ref[...] = jnp.zeros_like(o_ref)

    o_ref[...] += contrib


def _tc_pool(h, r, w, b, batch3):
    return pl.pallas_call(
        _pool_body,
        grid=(_GRID,),
        in_specs=[
            pl.BlockSpec((_ROW_BLK, D), lambda i: (i, 0)),
            pl.BlockSpec((1, _ROW_BLK, D), lambda i: (0, i, 0)),
            pl.BlockSpec((1, _ROW_BLK, D), lambda i: (1, i, 0)),
            pl.BlockSpec((D, D), lambda i: (0, 0)),
            pl.BlockSpec((1, D), lambda i: (0, 0)),
            pl.BlockSpec((1, 1, _ROW_BLK), lambda i: (i, 0, 0)),
        ],
        out_specs=pl.BlockSpec((B, D), lambda i: (0, 0)),
        out_shape=jax.ShapeDtypeStruct((B, D), jnp.float32),
    )(h, r, r, w, b.reshape(1, D), batch3)


def kernel(x, edge_index, batch, emb_table, W0, b0, W1, b1):
    x = x.astype(jnp.int32)
    tok_pad = EMB_PER_SC - NHALF * L
    halves_src = []
    halves_dst = []
    dst_local = jnp.repeat(jnp.arange(NHALF, dtype=jnp.int32), L)
    trash = NHALF + (jnp.arange(tok_pad, dtype=jnp.int32)
                     % (EMB_ACC_ROWS - NHALF))
    for c in range(NC):
        xs = x[c * NHALF:(c + 1) * NHALF].reshape(-1)
        halves_src.append(jnp.concatenate([xs, jnp.zeros((tok_pad,), jnp.int32)]))
        halves_dst.append(jnp.concatenate([dst_local, trash]))
    src0 = jnp.concatenate(halves_src)
    dst0 = jnp.concatenate(halves_dst)

    etrash = N + (jnp.arange(EP1 - E, dtype=jnp.int32) % (ACC_ROWS - N))
    src1 = jnp.concatenate(
        [edge_index[0].astype(jnp.int32), jnp.zeros((EP1 - E,), jnp.int32)])
    dst1 = jnp.concatenate([edge_index[1].astype(jnp.int32), etrash])

    zeros_blk = jnp.zeros((ROWS_PER_TILE, D), jnp.float32)
    batch3 = batch.astype(jnp.int32).reshape(_GRID, 1, _ROW_BLK)

    h0 = _emb_kernel(emb_table, src0, dst0, zeros_blk)
    q = _edge_kernel(h0, src1, dst1, zeros_blk).reshape(NC, ACC_ROWS, D)
    h1 = _tc_layer(h0, q, W0, b0)
    r = _edge_kernel(h1, src1, dst1, zeros_blk).reshape(NC, ACC_ROWS, D)
    return _tc_pool(h1, r, W1, b1, batch3)
